# baseline (speedup 1.0000x reference)
import functools

import jax
import jax.numpy as jnp
from jax import lax
from jax.experimental import pallas as pl
from jax.experimental.pallas import tpu as pltpu
from jax.experimental.pallas import tpu_sc as plsc

N = 10000
E = 320000
D = 128
DE = 16

BLK = 128
NBLK = E // BLK
BPW = NBLK // 32
CHUNK = 13
NCH = BPW // CHUNK
ECH = CHUNK * BLK
NPAD = 10240
RPT = NPAD // 16


def _sc_scatter_body(zeros_hbm, idx_hbm, attr_hbm, out_hbm,
                     idx_v, av0, av1, tr0, tr1, shared, lsem, ssem):
    cid = lax.axis_index("c")
    sid = lax.axis_index("s")
    w = sid * 2 + cid
    row0 = sid * RPT

    pltpu.sync_copy(zeros_hbm.at[pl.ds(row0, RPT)],
                    shared.at[pl.ds(row0, RPT)])
    plsc.subcore_barrier()

    base_b = w * BPW + jnp.minimum(w, 4)
    pltpu.sync_copy(idx_hbm.at[pl.ds(base_b, BPW), 0],
                    idx_v.at[pl.ds(0, BPW)])

    avs = [av0, av1]
    trs = [tr0, tr1]
    iota = lax.iota(jnp.int32, 16)
    fhi_vec = iota // 8
    flo_vec = iota % 8

    def transpose_chunk(av, tr, nblk):
        def blk_body(eblk, _):
            blk_vec = jnp.full((16,), eblk, jnp.int32)
            e0 = eblk * BLK

            @plsc.parallel_loop(0, BLK, step=1, unroll=16)
            def _t(elo):
                vals = plsc.load_gather(
                    av, [fhi_vec, blk_vec, flo_vec,
                         jnp.full((16,), elo, jnp.int32)])
                tr[e0 + elo] = vals

            return _

        lax.fori_loop(0, nblk, blk_body, 0)

    loads = [None] * NCH
    scatters = [None] * NCH
    loads[0] = pltpu.async_copy(
        attr_hbm.at[:, pl.ds(base_b, CHUNK)],
        avs[0].at[:, :, :, pl.ds(0, BLK)], lsem)
    for ch in range(NCH):
        buf = ch & 1
        if ch + 1 < NCH:
            loads[ch + 1] = pltpu.async_copy(
                attr_hbm.at[:, pl.ds(base_b + (ch + 1) * CHUNK, CHUNK)],
                avs[(ch + 1) & 1].at[:, :, :, pl.ds(0, BLK)], lsem)
        if ch >= 2:
            for dsc in scatters[ch - 2]:
                dsc.wait()
        loads[ch].wait()
        transpose_chunk(avs[buf], trs[buf], CHUNK)
        scatters[ch] = [
            pltpu.async_copy(trs[buf].at[pl.ds(b * BLK, BLK)],
                             shared.at[idx_v.at[ch * CHUNK + b]],
                             ssem, add=True)
            for b in range(CHUNK)
        ]
    for ch in (NCH - 2, NCH - 1):
        for dsc in scatters[ch]:
            dsc.wait()

    @pl.when(w < 4)
    def _tail():
        pltpu.sync_copy(idx_hbm.at[base_b + BPW, 0], idx_v.at[BPW])
        pltpu.sync_copy(attr_hbm.at[:, pl.ds(base_b + BPW, 1)],
                        avs[0].at[:, pl.ds(0, 1), :, pl.ds(0, BLK)])
        transpose_chunk(avs[0], trs[0], 1)
        pltpu.sync_copy(trs[0].at[pl.ds(0, BLK)],
                        shared.at[idx_v.at[BPW]], add=True)

    plsc.subcore_barrier()

    pltpu.sync_copy(shared.at[pl.ds(row0, RPT)],
                    out_hbm.at[cid, pl.ds(row0, RPT)])


_sc_scatter = functools.partial(
    pl.kernel,
    out_type=jax.ShapeDtypeStruct((2, NPAD, DE), jnp.float32),
    mesh=plsc.VectorSubcoreMesh(core_axis_name="c", subcore_axis_name="s"),
    scratch_types=[
        pltpu.VMEM((BPW + 1, BLK), jnp.int32),
        pltpu.VMEM((2, CHUNK, 8, BLK + 1), jnp.float32),
        pltpu.VMEM((2, CHUNK, 8, BLK + 1), jnp.float32),
        pltpu.VMEM((ECH, DE), jnp.float32),
        pltpu.VMEM((ECH, DE), jnp.float32),
        pltpu.VMEM_SHARED((NPAD, DE), jnp.float32),
        pltpu.SemaphoreType.DMA,
        pltpu.SemaphoreType.DMA,
    ],
    compiler_params=pltpu.CompilerParams(
        use_tc_tiling_on_sc=False, needs_layout_passes=False),
)(_sc_scatter_body)


def _tc_pre_body(x_ref, w1x_ref, b1_ref, o_ref):
    o_ref[...] = (jnp.dot(x_ref[...], w1x_ref[...],
                          preferred_element_type=jnp.float32) + b1_ref[...])


def _tc_pre(x, w1x, b1):
    rows = 1000
    grid = (N // rows,)
    return pl.pallas_call(
        _tc_pre_body,
        grid=grid,
        in_specs=[
            pl.BlockSpec((rows, D), lambda i: (i, 0)),
            pl.BlockSpec((D, D), lambda i: (0, 0)),
            pl.BlockSpec((1, D), lambda i: (0, 0)),
        ],
        out_specs=pl.BlockSpec((rows, D), lambda i: (i, 0)),
        out_shape=jax.ShapeDtypeStruct((N, D), jnp.float32),
    )(x, w1x, b1)


def _tc_mlp_body(x_ref, h1_ref, p0_ref, p1_ref, w1a_ref, w2_ref,
                 b2_ref, g_ref, bt_ref, o_ref):
    x = x_ref[...]
    pp = p0_ref[...] + p1_ref[...]
    ha = jnp.dot(pp, w1a_ref[...], preferred_element_type=jnp.float32)
    ha = ha.reshape(pp.shape[0], 8, D).reshape(x.shape[0], D)
    h = h1_ref[...] + ha
    h = h * jax.nn.sigmoid(h)
    h = jnp.dot(h, w2_ref[...], preferred_element_type=jnp.float32) + b2_ref[...]
    mu = jnp.mean(h, axis=-1, keepdims=True)
    var = jnp.mean((h - mu) ** 2, axis=-1, keepdims=True)
    h = (h - mu) * lax.rsqrt(var + 1e-5) * g_ref[...] + bt_ref[...]
    o_ref[...] = h + x


def _tc_mlp(x, h1, p0, p1, w1a, w2, b2, gamma, beta):
    rows = 640
    grid = ((N + rows - 1) // rows,)
    prows = rows // 8
    full = lambda shape: pl.BlockSpec(shape, lambda i: (0, 0))
    return pl.pallas_call(
        _tc_mlp_body,
        grid=grid,
        in_specs=[
            pl.BlockSpec((rows, D), lambda i: (i, 0)),
            pl.BlockSpec((rows, D), lambda i: (i, 0)),
            pl.BlockSpec((prows, D), lambda i: (i, 0)),
            pl.BlockSpec((prows, D), lambda i: (i + NPAD * DE // D // prows, 0)),
            full((D, 8 * D)),
            full((D, D)),
            full((1, D)),
            full((1, D)),
            full((1, D)),
        ],
        out_specs=pl.BlockSpec((rows, D), lambda i: (i, 0)),
        out_shape=jax.ShapeDtypeStruct((N, D), jnp.float32),
    )(x, h1, p0, p1, w1a, w2, b2, gamma, beta)


def kernel(x, edge_index, edge_attr, W1, b1, W2, b2, gamma, beta):
    idx_r = edge_index.reshape(2, NBLK, BLK).transpose(1, 0, 2)
    attr4 = edge_attr.T.reshape(2, 8, NBLK, BLK).transpose(0, 2, 1, 3)
    zeros = jnp.zeros((NPAD, DE), jnp.float32)

    partials = _sc_scatter(zeros, idx_r, attr4)
    p_packed = partials.reshape(2 * NPAD * DE // D, D)

    w1x = W1[:D]
    w1a = W1[D:]
    w1ap = jnp.kron(jnp.eye(8, dtype=jnp.float32), w1a)
    h1 = _tc_pre(x, w1x, b1.reshape(1, D))
    return _tc_mlp(x, h1, p_packed, p_packed, w1ap,
                   W2, b2.reshape(1, D),
                   gamma.reshape(1, D), beta.reshape(1, D))


# speedup vs baseline: 1.0321x; 1.0321x over previous
import functools

import jax
import jax.numpy as jnp
from jax import lax
from jax.experimental import pallas as pl
from jax.experimental.pallas import tpu as pltpu
from jax.experimental.pallas import tpu_sc as plsc

N = 10000
E = 320000
D = 128
DE = 16

BLK = 128
NBLK = E // BLK
BPW = NBLK // 32
CHUNK = 13
NCH = BPW // CHUNK
ECH = CHUNK * BLK
NPAD = 10240
RPT = NPAD // 16


def _sc_scatter_body(idx_hbm, attr_hbm, out_hbm,
                     idx_v, av0, av1, tr0, tr1, shared, lsem, ssem):
    cid = lax.axis_index("c")
    sid = lax.axis_index("s")
    w = sid * 2 + cid
    row0 = sid * RPT

    @plsc.parallel_loop(0, RPT, step=1, unroll=8)
    def _z(i):
        tr0[i] = jnp.zeros((16,), jnp.float32)

    pltpu.sync_copy(tr0.at[pl.ds(0, RPT)], shared.at[pl.ds(row0, RPT)])
    plsc.subcore_barrier()

    base_b = w * BPW + jnp.minimum(w, 4)
    pltpu.sync_copy(idx_hbm.at[pl.ds(base_b, BPW), 0],
                    idx_v.at[pl.ds(0, BPW)])

    avs = [av0, av1]
    trs = [tr0, tr1]
    iota = lax.iota(jnp.int32, 16)
    fhi_vec = iota // 8
    flo_vec = iota % 8

    def transpose_chunk(av, tr, nblk):
        def blk_body(eblk, _):
            blk_vec = jnp.full((16,), eblk, jnp.int32)
            e0 = eblk * BLK

            @plsc.parallel_loop(0, BLK, step=1, unroll=16)
            def _t(elo):
                vals = plsc.load_gather(
                    av, [fhi_vec, blk_vec, flo_vec,
                         jnp.full((16,), elo, jnp.int32)])
                tr[e0 + elo] = vals

            return _

        lax.fori_loop(0, nblk, blk_body, 0)

    loads = [None] * NCH
    scatters = [None] * NCH
    loads[0] = pltpu.async_copy(
        attr_hbm.at[:, pl.ds(base_b, CHUNK)],
        avs[0].at[:, :, :, pl.ds(0, BLK)], lsem)
    for ch in range(NCH):
        buf = ch & 1
        if ch + 1 < NCH:
            loads[ch + 1] = pltpu.async_copy(
                attr_hbm.at[:, pl.ds(base_b + (ch + 1) * CHUNK, CHUNK)],
                avs[(ch + 1) & 1].at[:, :, :, pl.ds(0, BLK)], lsem)
        if ch >= 2:
            for dsc in scatters[ch - 2]:
                dsc.wait()
        loads[ch].wait()
        transpose_chunk(avs[buf], trs[buf], CHUNK)
        scatters[ch] = [
            pltpu.async_copy(trs[buf].at[pl.ds(b * BLK, BLK)],
                             shared.at[idx_v.at[ch * CHUNK + b]],
                             ssem, add=True)
            for b in range(CHUNK)
        ]
    for ch in (NCH - 2, NCH - 1):
        for dsc in scatters[ch]:
            dsc.wait()

    @pl.when(w < 4)
    def _tail():
        pltpu.sync_copy(idx_hbm.at[base_b + BPW, 0], idx_v.at[BPW])
        pltpu.sync_copy(attr_hbm.at[:, pl.ds(base_b + BPW, 1)],
                        avs[0].at[:, pl.ds(0, 1), :, pl.ds(0, BLK)])
        transpose_chunk(avs[0], trs[0], 1)
        pltpu.sync_copy(trs[0].at[pl.ds(0, BLK)],
                        shared.at[idx_v.at[BPW]], add=True)

    plsc.subcore_barrier()

    pltpu.sync_copy(shared.at[pl.ds(row0, RPT)],
                    out_hbm.at[cid, pl.ds(row0, RPT)])


_sc_scatter = functools.partial(
    pl.kernel,
    out_type=jax.ShapeDtypeStruct((2, NPAD, DE), jnp.float32),
    mesh=plsc.VectorSubcoreMesh(core_axis_name="c", subcore_axis_name="s"),
    scratch_types=[
        pltpu.VMEM((BPW + 1, BLK), jnp.int32),
        pltpu.VMEM((2, CHUNK, 8, BLK + 1), jnp.float32),
        pltpu.VMEM((2, CHUNK, 8, BLK + 1), jnp.float32),
        pltpu.VMEM((ECH, DE), jnp.float32),
        pltpu.VMEM((ECH, DE), jnp.float32),
        pltpu.VMEM_SHARED((NPAD, DE), jnp.float32),
        pltpu.SemaphoreType.DMA,
        pltpu.SemaphoreType.DMA,
    ],
    compiler_params=pltpu.CompilerParams(
        use_tc_tiling_on_sc=False, needs_layout_passes=False),
)(_sc_scatter_body)


def _tc_mlp_body(x_ref, p0_ref, p1_ref, w1x_ref, w1a_ref, b1_ref, w2_ref,
                 b2_ref, g_ref, bt_ref, o_ref):
    x = x_ref[...]
    pp = p0_ref[...] + p1_ref[...]
    ha = jnp.dot(pp, w1a_ref[...], preferred_element_type=jnp.float32)
    ha = ha.reshape(pp.shape[0], 8, D).reshape(x.shape[0], D)
    h = (jnp.dot(x, w1x_ref[...], preferred_element_type=jnp.float32)
         + ha + b1_ref[...])
    h = h * jax.nn.sigmoid(h)
    h = jnp.dot(h, w2_ref[...], preferred_element_type=jnp.float32) + b2_ref[...]
    mu = jnp.mean(h, axis=-1, keepdims=True)
    var = jnp.mean((h - mu) ** 2, axis=-1, keepdims=True)
    h = (h - mu) * lax.rsqrt(var + 1e-5) * g_ref[...] + bt_ref[...]
    o_ref[...] = h + x


def _tc_mlp(x, p0, p1, w1x, w1a, b1, w2, b2, gamma, beta):
    rows = 640
    grid = ((N + rows - 1) // rows,)
    prows = rows // 8
    full = lambda shape: pl.BlockSpec(shape, lambda i: (0, 0))
    return pl.pallas_call(
        _tc_mlp_body,
        grid=grid,
        in_specs=[
            pl.BlockSpec((rows, D), lambda i: (i, 0)),
            pl.BlockSpec((prows, D), lambda i: (i, 0)),
            pl.BlockSpec((prows, D), lambda i: (i + NPAD * DE // D // prows, 0)),
            full((D, D)),
            full((D, 8 * D)),
            full((1, D)),
            full((D, D)),
            full((1, D)),
            full((1, D)),
            full((1, D)),
        ],
        out_specs=pl.BlockSpec((rows, D), lambda i: (i, 0)),
        out_shape=jax.ShapeDtypeStruct((N, D), jnp.float32),
    )(x, p0, p1, w1x, w1a, b1, w2, b2, gamma, beta)


def kernel(x, edge_index, edge_attr, W1, b1, W2, b2, gamma, beta):
    idx_r = edge_index.reshape(2, NBLK, BLK).transpose(1, 0, 2)
    attr4 = edge_attr.T.reshape(2, 8, NBLK, BLK).transpose(0, 2, 1, 3)

    partials = _sc_scatter(idx_r, attr4)
    p_packed = partials.reshape(2 * NPAD * DE // D, D)

    w1x = W1[:D]
    w1a = W1[D:]
    w1ap = jnp.kron(jnp.eye(8, dtype=jnp.float32), w1a)
    return _tc_mlp(x, p_packed, p_packed, w1x, w1ap,
                   b1.reshape(1, D), W2, b2.reshape(1, D),
                   gamma.reshape(1, D), beta.reshape(1, D))


# speedup vs baseline: 1.0808x; 1.0471x over previous
import functools

import jax
import jax.numpy as jnp
from jax import lax
from jax.experimental import pallas as pl
from jax.experimental.pallas import tpu as pltpu
from jax.experimental.pallas import tpu_sc as plsc

N = 10000
E = 320000
D = 128
DE = 16

BLK = 128
NBLK = E // BLK
BPW = NBLK // 32
CHUNK = 13
NCH = BPW // CHUNK
ECH = CHUNK * BLK
NPAD = 10240
RPT = NPAD // 16


def _sc_scatter_body(idx_hbm, attr_hbm, out_hbm,
                     idx_v, av0, av1, tr0, tr1, shared, lsem, ssem):
    cid = lax.axis_index("c")
    sid = lax.axis_index("s")
    w = sid * 2 + cid
    row0 = sid * RPT

    @plsc.parallel_loop(0, RPT, step=1, unroll=8)
    def _z(i):
        tr0[i] = jnp.zeros((16,), jnp.float32)

    pltpu.sync_copy(tr0.at[pl.ds(0, RPT)], shared.at[pl.ds(row0, RPT)])
    plsc.subcore_barrier()

    base_b = w * BPW + jnp.minimum(w, 4)
    pltpu.sync_copy(idx_hbm.at[pl.ds(base_b, BPW), 0],
                    idx_v.at[pl.ds(0, BPW)])

    avs = [av0, av1]
    trs = [tr0, tr1]
    iota = lax.iota(jnp.int32, 16)
    fhi_vec = iota // 8
    flo_vec = iota % 8

    def transpose_chunk(av, tr, nblk):
        def blk_body(eblk, _):
            blk_vec = jnp.full((16,), eblk, jnp.int32)
            e0 = eblk * BLK

            @plsc.parallel_loop(0, BLK, step=1, unroll=16)
            def _t(elo):
                vals = plsc.load_gather(
                    av, [fhi_vec, blk_vec, flo_vec,
                         jnp.full((16,), elo, jnp.int32)])
                tr[e0 + elo] = vals

            return _

        lax.fori_loop(0, nblk, blk_body, 0)

    loads = [None] * NCH
    scatters = [None] * NCH
    loads[0] = pltpu.async_copy(
        attr_hbm.at[:, pl.ds(base_b, CHUNK)],
        avs[0].at[:, :, :, pl.ds(0, BLK)], lsem)
    for ch in range(NCH):
        buf = ch & 1
        if ch + 1 < NCH:
            loads[ch + 1] = pltpu.async_copy(
                attr_hbm.at[:, pl.ds(base_b + (ch + 1) * CHUNK, CHUNK)],
                avs[(ch + 1) & 1].at[:, :, :, pl.ds(0, BLK)], lsem)
        if ch >= 2:
            for dsc in scatters[ch - 2]:
                dsc.wait()
        loads[ch].wait()
        transpose_chunk(avs[buf], trs[buf], CHUNK)
        scatters[ch] = [
            pltpu.async_copy(trs[buf].at[pl.ds(b * BLK, BLK)],
                             shared.at[idx_v.at[ch * CHUNK + b]],
                             ssem, add=True)
            for b in range(CHUNK)
        ]
    for ch in (NCH - 2, NCH - 1):
        for dsc in scatters[ch]:
            dsc.wait()

    @pl.when(w < 4)
    def _tail():
        pltpu.sync_copy(idx_hbm.at[base_b + BPW, 0], idx_v.at[BPW])
        pltpu.sync_copy(attr_hbm.at[:, pl.ds(base_b + BPW, 1)],
                        avs[0].at[:, pl.ds(0, 1), :, pl.ds(0, BLK)])
        transpose_chunk(avs[0], trs[0], 1)
        pltpu.sync_copy(trs[0].at[pl.ds(0, BLK)],
                        shared.at[idx_v.at[BPW]], add=True)

    plsc.subcore_barrier()

    pltpu.sync_copy(shared.at[pl.ds(row0, RPT)],
                    out_hbm.at[cid, pl.ds(row0, RPT)])


_sc_scatter = functools.partial(
    pl.kernel,
    out_type=jax.ShapeDtypeStruct((2, NPAD, DE), jnp.float32),
    mesh=plsc.VectorSubcoreMesh(core_axis_name="c", subcore_axis_name="s"),
    scratch_types=[
        pltpu.VMEM((BPW + 1, BLK), jnp.int32),
        pltpu.VMEM((2, CHUNK, 8, BLK + 1), jnp.float32),
        pltpu.VMEM((2, CHUNK, 8, BLK + 1), jnp.float32),
        pltpu.VMEM((ECH, DE), jnp.float32),
        pltpu.VMEM((ECH, DE), jnp.float32),
        pltpu.VMEM_SHARED((NPAD, DE), jnp.float32),
        pltpu.SemaphoreType.DMA,
        pltpu.SemaphoreType.DMA,
    ],
    compiler_params=pltpu.CompilerParams(
        use_tc_tiling_on_sc=False, needs_layout_passes=False),
)(_sc_scatter_body)


def _tc_mlp_body(x_ref, p0_ref, p1_ref, w1x_ref, w1a_ref, b1_ref, w2_ref,
                 b2_ref, g_ref, bt_ref, o_ref):
    x = x_ref[...]
    pp = p0_ref[...] + p1_ref[...]
    ha = jnp.dot(pp, w1a_ref[...], preferred_element_type=jnp.float32)
    ha = ha.reshape(pp.shape[0], 8, D).reshape(x.shape[0], D)
    h = (jnp.dot(x, w1x_ref[...], preferred_element_type=jnp.float32)
         + ha + b1_ref[...])
    h = h * jax.nn.sigmoid(h)
    h = jnp.dot(h, w2_ref[...], preferred_element_type=jnp.float32) + b2_ref[...]
    mu = jnp.mean(h, axis=-1, keepdims=True)
    var = jnp.mean((h - mu) ** 2, axis=-1, keepdims=True)
    h = (h - mu) * lax.rsqrt(var + 1e-5) * g_ref[...] + bt_ref[...]
    o_ref[...] = h + x


def _tc_mlp(x, p0, p1, w1x, w1a, b1, w2, b2, gamma, beta):
    rows = 960
    grid = ((N + rows - 1) // rows,)
    prows = rows // 8
    full = lambda shape: pl.BlockSpec(shape, lambda i: (0, 0))
    return pl.pallas_call(
        _tc_mlp_body,
        grid=grid,
        in_specs=[
            pl.BlockSpec((rows, D), lambda i: (i, 0)),
            pl.BlockSpec((prows, D), lambda i: (i, 0)),
            pl.BlockSpec((prows, D), lambda i: (i + NPAD * DE // D // prows, 0)),
            full((D, D)),
            full((D, 8 * D)),
            full((1, D)),
            full((D, D)),
            full((1, D)),
            full((1, D)),
            full((1, D)),
        ],
        out_specs=pl.BlockSpec((rows, D), lambda i: (i, 0)),
        out_shape=jax.ShapeDtypeStruct((N, D), jnp.float32),
    )(x, p0, p1, w1x, w1a, b1, w2, b2, gamma, beta)


def kernel(x, edge_index, edge_attr, W1, b1, W2, b2, gamma, beta):
    idx_r = edge_index.reshape(2, NBLK, BLK).transpose(1, 0, 2)
    attr4 = edge_attr.T.reshape(2, 8, NBLK, BLK).transpose(0, 2, 1, 3)

    partials = _sc_scatter(idx_r, attr4)
    p_packed = partials.reshape(2 * NPAD * DE // D, D)

    w1x = W1[:D]
    w1a = W1[D:]
    w1ap = jnp.kron(jnp.eye(8, dtype=jnp.float32), w1a)
    return _tc_mlp(x, p_packed, p_packed, w1x, w1ap,
                   b1.reshape(1, D), W2, b2.reshape(1, D),
                   gamma.reshape(1, D), beta.reshape(1, D))


# speedup vs baseline: 1.0942x; 1.0124x over previous
import functools

import jax
import jax.numpy as jnp
from jax import lax
from jax.experimental import pallas as pl
from jax.experimental.pallas import tpu as pltpu
from jax.experimental.pallas import tpu_sc as plsc

N = 10000
E = 320000
D = 128
DE = 16

BLK = 128
NBLK = E // BLK
BPW = NBLK // 32
CHUNK = 13
NCH = BPW // CHUNK
ECH = CHUNK * BLK
NPAD = 10240
RPT = NPAD // 16


def _sc_scatter_body(idx_hbm, attr_hbm, out_hbm,
                     idx_v, av0, av1, tr0, tr1, shared, lsem, ssem):
    cid = lax.axis_index("c")
    sid = lax.axis_index("s")
    w = sid * 2 + cid
    row0 = sid * RPT

    @plsc.parallel_loop(0, RPT, step=1, unroll=8)
    def _z(i):
        tr0[i] = jnp.zeros((16,), jnp.float32)

    pltpu.sync_copy(tr0.at[pl.ds(0, RPT)], shared.at[pl.ds(row0, RPT)])
    plsc.subcore_barrier()

    base_b = w * BPW + jnp.minimum(w, 4)
    pltpu.sync_copy(idx_hbm.at[pl.ds(base_b, BPW), 0],
                    idx_v.at[pl.ds(0, BPW)])

    avs = [av0, av1]
    trs = [tr0, tr1]
    iota = lax.iota(jnp.int32, 16)
    fhi_vec = iota // 8
    flo_vec = iota % 8

    def transpose_chunk(av, tr, nblk):
        def blk_body(eblk, _):
            blk_vec = jnp.full((16,), eblk, jnp.int32)
            e0 = eblk * BLK

            @plsc.parallel_loop(0, BLK, step=1, unroll=16)
            def _t(elo):
                vals = plsc.load_gather(
                    av, [fhi_vec, blk_vec, flo_vec,
                         jnp.full((16,), elo, jnp.int32)])
                tr[e0 + elo] = vals

            return _

        lax.fori_loop(0, nblk, blk_body, 0)

    loads = [None] * NCH
    scatters = [None] * NCH
    loads[0] = pltpu.async_copy(
        attr_hbm.at[:, pl.ds(base_b, CHUNK)],
        avs[0].at[:, :, :, pl.ds(0, BLK)], lsem)
    for ch in range(NCH):
        buf = ch & 1
        if ch + 1 < NCH:
            loads[ch + 1] = pltpu.async_copy(
                attr_hbm.at[:, pl.ds(base_b + (ch + 1) * CHUNK, CHUNK)],
                avs[(ch + 1) & 1].at[:, :, :, pl.ds(0, BLK)], lsem)
        if ch >= 2:
            for dsc in scatters[ch - 2]:
                dsc.wait()
        loads[ch].wait()
        transpose_chunk(avs[buf], trs[buf], CHUNK)
        scatters[ch] = [
            pltpu.async_copy(trs[buf].at[pl.ds(b * BLK, BLK)],
                             shared.at[idx_v.at[ch * CHUNK + b]],
                             ssem, add=True)
            for b in range(CHUNK)
        ]
    for ch in (NCH - 2, NCH - 1):
        for dsc in scatters[ch]:
            dsc.wait()

    @pl.when(w < 4)
    def _tail():
        pltpu.sync_copy(idx_hbm.at[base_b + BPW, 0], idx_v.at[BPW])
        pltpu.sync_copy(attr_hbm.at[:, pl.ds(base_b + BPW, 1)],
                        avs[0].at[:, pl.ds(0, 1), :, pl.ds(0, BLK)])
        transpose_chunk(avs[0], trs[0], 1)
        pltpu.sync_copy(trs[0].at[pl.ds(0, BLK)],
                        shared.at[idx_v.at[BPW]], add=True)

    plsc.subcore_barrier()

    pltpu.sync_copy(shared.at[pl.ds(row0, RPT)],
                    out_hbm.at[cid, pl.ds(row0, RPT)])


_sc_scatter = functools.partial(
    pl.kernel,
    out_type=jax.ShapeDtypeStruct((2, NPAD, DE), jnp.float32),
    mesh=plsc.VectorSubcoreMesh(core_axis_name="c", subcore_axis_name="s"),
    scratch_types=[
        pltpu.VMEM((BPW + 1, BLK), jnp.int32),
        pltpu.VMEM((2, CHUNK, 8, BLK + 1), jnp.float32),
        pltpu.VMEM((2, CHUNK, 8, BLK + 1), jnp.float32),
        pltpu.VMEM((ECH, DE), jnp.float32),
        pltpu.VMEM((ECH, DE), jnp.float32),
        pltpu.VMEM_SHARED((NPAD, DE), jnp.float32),
        pltpu.SemaphoreType.DMA,
        pltpu.SemaphoreType.DMA,
    ],
    compiler_params=pltpu.CompilerParams(
        use_tc_tiling_on_sc=False, needs_layout_passes=False),
)(_sc_scatter_body)


def _tc_mlp_body(x_ref, p0_ref, p1_ref, w1x_ref, w1a_ref, b1_ref, w2_ref,
                 b2_ref, g_ref, bt_ref, o_ref):
    x = x_ref[...]
    pp = p0_ref[...] + p1_ref[...]
    ha = jnp.dot(pp, w1a_ref[...], preferred_element_type=jnp.float32)
    ha = ha.reshape(pp.shape[0], 8, D).reshape(x.shape[0], D)
    h = (jnp.dot(x, w1x_ref[...], preferred_element_type=jnp.float32)
         + ha + b1_ref[...])
    h = h * jax.nn.sigmoid(h)
    h = jnp.dot(h, w2_ref[...], preferred_element_type=jnp.float32) + b2_ref[...]
    mu = jnp.mean(h, axis=-1, keepdims=True)
    var = jnp.mean((h - mu) ** 2, axis=-1, keepdims=True)
    h = (h - mu) * lax.rsqrt(var + 1e-5) * g_ref[...] + bt_ref[...]
    o_ref[...] = h + x


def _tc_mlp(x, p0, p1, w1x, w1a, b1, w2, b2, gamma, beta):
    rows = 1024
    grid = ((N + rows - 1) // rows,)
    prows = rows // 8
    full = lambda shape: pl.BlockSpec(shape, lambda i: (0, 0))
    return pl.pallas_call(
        _tc_mlp_body,
        grid=grid,
        in_specs=[
            pl.BlockSpec((rows, D), lambda i: (i, 0)),
            pl.BlockSpec((prows, D), lambda i: (i, 0)),
            pl.BlockSpec((prows, D), lambda i: (i + NPAD * DE // D // prows, 0)),
            full((D, D)),
            full((D, 8 * D)),
            full((1, D)),
            full((D, D)),
            full((1, D)),
            full((1, D)),
            full((1, D)),
        ],
        out_specs=pl.BlockSpec((rows, D), lambda i: (i, 0)),
        out_shape=jax.ShapeDtypeStruct((N, D), jnp.float32),
    )(x, p0, p1, w1x, w1a, b1, w2, b2, gamma, beta)


def kernel(x, edge_index, edge_attr, W1, b1, W2, b2, gamma, beta):
    idx_r = edge_index.reshape(2, NBLK, BLK).transpose(1, 0, 2)
    attr4 = edge_attr.T.reshape(2, 8, NBLK, BLK).transpose(0, 2, 1, 3)

    partials = _sc_scatter(idx_r, attr4)
    p_packed = partials.reshape(2 * NPAD * DE // D, D)

    w1x = W1[:D]
    w1a = W1[D:]
    w1ap = jnp.kron(jnp.eye(8, dtype=jnp.float32), w1a)
    return _tc_mlp(x, p_packed, p_packed, w1x, w1ap,
                   b1.reshape(1, D), W2, b2.reshape(1, D),
                   gamma.reshape(1, D), beta.reshape(1, D))


# speedup vs baseline: 1.1136x; 1.0178x over previous
import functools

import jax
import jax.numpy as jnp
from jax import lax
from jax.experimental import pallas as pl
from jax.experimental.pallas import tpu as pltpu
from jax.experimental.pallas import tpu_sc as plsc

N = 10000
E = 320000
D = 128
DE = 16

BLK = 128
NBLK = E // BLK
BPW = NBLK // 32
CHUNK = 13
NCH = BPW // CHUNK
ECH = CHUNK * BLK
NPAD = 10240
RPT = NPAD // 16


def _sc_scatter_body(idx_hbm, attr_hbm, out_hbm,
                     idx_v, av0, av1, tr0, tr1, shared, lsem, ssem):
    cid = lax.axis_index("c")
    sid = lax.axis_index("s")
    w = sid * 2 + cid
    row0 = sid * RPT

    @plsc.parallel_loop(0, RPT, step=1, unroll=8)
    def _z(i):
        tr0[i] = jnp.zeros((16,), jnp.float32)

    pltpu.sync_copy(tr0.at[pl.ds(0, RPT)], shared.at[pl.ds(row0, RPT)])
    plsc.subcore_barrier()

    base_b = w * BPW + jnp.minimum(w, 4)
    pltpu.sync_copy(idx_hbm.at[pl.ds(base_b, BPW), 0],
                    idx_v.at[pl.ds(0, BPW)])

    avs = [av0, av1]
    trs = [tr0, tr1]
    iota = lax.iota(jnp.int32, 16)
    fhi_vec = iota // 8
    flo_vec = iota % 8

    def transpose_chunk(av, tr, nblk):
        def blk_body(eblk, _):
            blk_vec = jnp.full((16,), eblk, jnp.int32)
            e0 = eblk * BLK

            @plsc.parallel_loop(0, BLK, step=1, unroll=16)
            def _t(elo):
                vals = plsc.load_gather(
                    av, [fhi_vec, blk_vec, flo_vec,
                         jnp.full((16,), elo, jnp.int32)])
                tr[e0 + elo] = vals

            return _

        lax.fori_loop(0, nblk, blk_body, 0)

    loads = [None] * NCH
    scatters = [None] * NCH
    loads[0] = pltpu.async_copy(
        attr_hbm.at[:, pl.ds(base_b, CHUNK)],
        avs[0].at[:, :, :, pl.ds(0, BLK)], lsem)
    for ch in range(NCH):
        buf = ch & 1
        if ch + 1 < NCH:
            loads[ch + 1] = pltpu.async_copy(
                attr_hbm.at[:, pl.ds(base_b + (ch + 1) * CHUNK, CHUNK)],
                avs[(ch + 1) & 1].at[:, :, :, pl.ds(0, BLK)], lsem)
        if ch >= 2:
            for dsc in scatters[ch - 2]:
                dsc.wait()
        loads[ch].wait()
        transpose_chunk(avs[buf], trs[buf], CHUNK)
        scatters[ch] = [
            pltpu.async_copy(trs[buf].at[pl.ds(b * BLK, BLK)],
                             shared.at[idx_v.at[ch * CHUNK + b]],
                             ssem, add=True)
            for b in range(CHUNK)
        ]
    for ch in (NCH - 2, NCH - 1):
        for dsc in scatters[ch]:
            dsc.wait()

    @pl.when(w < 4)
    def _tail():
        pltpu.sync_copy(idx_hbm.at[base_b + BPW, 0], idx_v.at[BPW])
        pltpu.sync_copy(attr_hbm.at[:, pl.ds(base_b + BPW, 1)],
                        avs[0].at[:, pl.ds(0, 1), :, pl.ds(0, BLK)])
        transpose_chunk(avs[0], trs[0], 1)
        pltpu.sync_copy(trs[0].at[pl.ds(0, BLK)],
                        shared.at[idx_v.at[BPW]], add=True)

    plsc.subcore_barrier()

    pltpu.sync_copy(shared.at[pl.ds(row0, RPT)],
                    out_hbm.at[cid, pl.ds(row0, RPT)])


_sc_scatter = functools.partial(
    pl.kernel,
    out_type=jax.ShapeDtypeStruct((2, NPAD, DE), jnp.float32),
    mesh=plsc.VectorSubcoreMesh(core_axis_name="c", subcore_axis_name="s"),
    scratch_types=[
        pltpu.VMEM((BPW + 1, BLK), jnp.int32),
        pltpu.VMEM((2, CHUNK, 8, BLK + 1), jnp.float32),
        pltpu.VMEM((2, CHUNK, 8, BLK + 1), jnp.float32),
        pltpu.VMEM((ECH, DE), jnp.float32),
        pltpu.VMEM((ECH, DE), jnp.float32),
        pltpu.VMEM_SHARED((NPAD, DE), jnp.float32),
        pltpu.SemaphoreType.DMA,
        pltpu.SemaphoreType.DMA,
    ],
    compiler_params=pltpu.CompilerParams(
        use_tc_tiling_on_sc=False, needs_layout_passes=False),
)(_sc_scatter_body)


def _tc_mlp_body(x_ref, p0_ref, p1_ref, w1x_ref, w1a_ref, b1_ref, w2_ref,
                 b2_ref, g_ref, bt_ref, o_ref):
    x = x_ref[...]
    pp = p0_ref[...] + p1_ref[...]
    ha = jnp.dot(pp, w1a_ref[...], preferred_element_type=jnp.float32)
    ha = ha.reshape(pp.shape[0], 8, D).reshape(x.shape[0], D)
    h = (jnp.dot(x, w1x_ref[...], preferred_element_type=jnp.float32)
         + ha + b1_ref[...])
    h = h * jax.nn.sigmoid(h)
    h = jnp.dot(h, w2_ref[...], preferred_element_type=jnp.float32) + b2_ref[...]
    mu = jnp.mean(h, axis=-1, keepdims=True)
    var = jnp.mean((h - mu) ** 2, axis=-1, keepdims=True)
    h = (h - mu) * lax.rsqrt(var + 1e-5) * g_ref[...] + bt_ref[...]
    o_ref[...] = h + x


def _tc_mlp(x, p0, p1, w1x, w1a, b1, w2, b2, gamma, beta):
    rows = 1280
    grid = ((N + rows - 1) // rows,)
    prows = rows // 8
    full = lambda shape: pl.BlockSpec(shape, lambda i: (0, 0))
    return pl.pallas_call(
        _tc_mlp_body,
        grid=grid,
        in_specs=[
            pl.BlockSpec((rows, D), lambda i: (i, 0)),
            pl.BlockSpec((prows, D), lambda i: (i, 0)),
            pl.BlockSpec((prows, D), lambda i: (i + NPAD * DE // D // prows, 0)),
            full((D, D)),
            full((D, 8 * D)),
            full((1, D)),
            full((D, D)),
            full((1, D)),
            full((1, D)),
            full((1, D)),
        ],
        out_specs=pl.BlockSpec((rows, D), lambda i: (i, 0)),
        out_shape=jax.ShapeDtypeStruct((N, D), jnp.float32),
    )(x, p0, p1, w1x, w1a, b1, w2, b2, gamma, beta)


def kernel(x, edge_index, edge_attr, W1, b1, W2, b2, gamma, beta):
    idx_r = edge_index.reshape(2, NBLK, BLK).transpose(1, 0, 2)
    attr4 = edge_attr.T.reshape(2, 8, NBLK, BLK).transpose(0, 2, 1, 3)

    partials = _sc_scatter(idx_r, attr4)
    p_packed = partials.reshape(2 * NPAD * DE // D, D)

    w1x = W1[:D]
    w1a = W1[D:]
    w1ap = jnp.kron(jnp.eye(8, dtype=jnp.float32), w1a)
    return _tc_mlp(x, p_packed, p_packed, w1x, w1ap,
                   b1.reshape(1, D), W2, b2.reshape(1, D),
                   gamma.reshape(1, D), beta.reshape(1, D))


# speedup vs baseline: 1.1515x; 1.0340x over previous
import functools

import jax
import jax.numpy as jnp
from jax import lax
from jax.experimental import pallas as pl
from jax.experimental.pallas import tpu as pltpu
from jax.experimental.pallas import tpu_sc as plsc

N = 10000
E = 320000
D = 128
DE = 16

BLK = 128
NBLK = E // BLK
BPW = NBLK // 32
CHUNK = 13
NCH = BPW // CHUNK
ECH = CHUNK * BLK
NPAD = 10240
RPT = NPAD // 16


def _sc_scatter_body(idx_hbm, attr_hbm, out_hbm,
                     idx_v, av0, av1, tr0, tr1, shared, lsem, ssem):
    cid = lax.axis_index("c")
    sid = lax.axis_index("s")
    w = sid * 2 + cid
    row0 = sid * RPT

    @plsc.parallel_loop(0, RPT, step=1, unroll=8)
    def _z(i):
        tr0[i] = jnp.zeros((16,), jnp.float32)

    pltpu.sync_copy(tr0.at[pl.ds(0, RPT)], shared.at[pl.ds(row0, RPT)])
    plsc.subcore_barrier()

    base_b = w * BPW + jnp.minimum(w, 4)
    pltpu.sync_copy(idx_hbm.at[pl.ds(base_b, BPW), 0],
                    idx_v.at[pl.ds(0, BPW)])

    avs = [av0, av1]
    trs = [tr0, tr1]
    iota = lax.iota(jnp.int32, 16)
    fhi_vec = iota // 8
    flo_vec = iota % 8

    def transpose_chunk(av, tr, nblk):
        def blk_body(eblk, _):
            blk_vec = jnp.full((16,), eblk, jnp.int32)
            e0 = eblk * BLK

            @plsc.parallel_loop(0, BLK, step=1, unroll=16)
            def _t(elo):
                vals = plsc.load_gather(
                    av, [fhi_vec, blk_vec, flo_vec,
                         jnp.full((16,), elo, jnp.int32)])
                tr[e0 + elo] = vals

            return _

        lax.fori_loop(0, nblk, blk_body, 0)

    loads = [None] * NCH
    scatters = [None] * NCH
    loads[0] = pltpu.async_copy(
        attr_hbm.at[:, pl.ds(base_b, CHUNK)],
        avs[0].at[:, :, :, pl.ds(0, BLK)], lsem)
    for ch in range(NCH):
        buf = ch & 1
        if ch + 1 < NCH:
            loads[ch + 1] = pltpu.async_copy(
                attr_hbm.at[:, pl.ds(base_b + (ch + 1) * CHUNK, CHUNK)],
                avs[(ch + 1) & 1].at[:, :, :, pl.ds(0, BLK)], lsem)
        if ch >= 2:
            for dsc in scatters[ch - 2]:
                dsc.wait()
        loads[ch].wait()
        transpose_chunk(avs[buf], trs[buf], CHUNK)
        scatters[ch] = [
            pltpu.async_copy(trs[buf].at[pl.ds(b * BLK, BLK)],
                             shared.at[idx_v.at[ch * CHUNK + b]],
                             ssem, add=True)
            for b in range(CHUNK)
        ]
    for ch in (NCH - 2, NCH - 1):
        for dsc in scatters[ch]:
            dsc.wait()

    @pl.when(w < 4)
    def _tail():
        pltpu.sync_copy(idx_hbm.at[base_b + BPW, 0], idx_v.at[BPW])
        pltpu.sync_copy(attr_hbm.at[:, pl.ds(base_b + BPW, 1)],
                        avs[0].at[:, pl.ds(0, 1), :, pl.ds(0, BLK)])
        transpose_chunk(avs[0], trs[0], 1)
        pltpu.sync_copy(trs[0].at[pl.ds(0, BLK)],
                        shared.at[idx_v.at[BPW]], add=True)

    plsc.subcore_barrier()

    pltpu.sync_copy(shared.at[pl.ds(row0, RPT)],
                    out_hbm.at[cid, pl.ds(row0, RPT)])


_sc_scatter = functools.partial(
    pl.kernel,
    out_type=jax.ShapeDtypeStruct((2, NPAD, DE), jnp.float32),
    mesh=plsc.VectorSubcoreMesh(core_axis_name="c", subcore_axis_name="s"),
    scratch_types=[
        pltpu.VMEM((BPW + 1, BLK), jnp.int32),
        pltpu.VMEM((2, CHUNK, 8, BLK + 1), jnp.float32),
        pltpu.VMEM((2, CHUNK, 8, BLK + 1), jnp.float32),
        pltpu.VMEM((ECH, DE), jnp.float32),
        pltpu.VMEM((ECH, DE), jnp.float32),
        pltpu.VMEM_SHARED((NPAD, DE), jnp.float32),
        pltpu.SemaphoreType.DMA,
        pltpu.SemaphoreType.DMA,
    ],
    compiler_params=pltpu.CompilerParams(
        use_tc_tiling_on_sc=False, needs_layout_passes=False),
)(_sc_scatter_body)


def _tc_mlp_body(x_ref, p0_ref, p1_ref, w1x_ref, w1a_ref, b1_ref, w2_ref,
                 b2_ref, g_ref, bt_ref, o_ref):
    x = x_ref[...]
    pp = p0_ref[...] + p1_ref[...]
    ha = jnp.dot(pp, w1a_ref[...], preferred_element_type=jnp.float32)
    ha = ha.reshape(pp.shape[0], 8, D).reshape(x.shape[0], D)
    h = (jnp.dot(x, w1x_ref[...], preferred_element_type=jnp.float32)
         + ha + b1_ref[...])
    h = h * jax.nn.sigmoid(h)
    h = jnp.dot(h, w2_ref[...], preferred_element_type=jnp.float32) + b2_ref[...]
    mu = jnp.mean(h, axis=-1, keepdims=True)
    var = jnp.mean((h - mu) ** 2, axis=-1, keepdims=True)
    h = (h - mu) * lax.rsqrt(var + 1e-5) * g_ref[...] + bt_ref[...]
    o_ref[...] = h + x


def _tc_mlp(x, p0, p1, w1x, w1a, b1, w2, b2, gamma, beta):
    rows = 2560
    grid = ((N + rows - 1) // rows,)
    prows = rows // 8
    full = lambda shape: pl.BlockSpec(shape, lambda i: (0, 0))
    return pl.pallas_call(
        _tc_mlp_body,
        grid=grid,
        in_specs=[
            pl.BlockSpec((rows, D), lambda i: (i, 0)),
            pl.BlockSpec((prows, D), lambda i: (i, 0)),
            pl.BlockSpec((prows, D), lambda i: (i + NPAD * DE // D // prows, 0)),
            full((D, D)),
            full((D, 8 * D)),
            full((1, D)),
            full((D, D)),
            full((1, D)),
            full((1, D)),
            full((1, D)),
        ],
        out_specs=pl.BlockSpec((rows, D), lambda i: (i, 0)),
        out_shape=jax.ShapeDtypeStruct((N, D), jnp.float32),
    )(x, p0, p1, w1x, w1a, b1, w2, b2, gamma, beta)


def kernel(x, edge_index, edge_attr, W1, b1, W2, b2, gamma, beta):
    idx_r = edge_index.reshape(2, NBLK, BLK).transpose(1, 0, 2)
    attr4 = edge_attr.T.reshape(2, 8, NBLK, BLK).transpose(0, 2, 1, 3)

    partials = _sc_scatter(idx_r, attr4)
    p_packed = partials.reshape(2 * NPAD * DE // D, D)

    w1x = W1[:D]
    w1a = W1[D:]
    w1ap = jnp.kron(jnp.eye(8, dtype=jnp.float32), w1a)
    return _tc_mlp(x, p_packed, p_packed, w1x, w1ap,
                   b1.reshape(1, D), W2, b2.reshape(1, D),
                   gamma.reshape(1, D), beta.reshape(1, D))


# speedup vs baseline: 1.1531x; 1.0014x over previous
import functools

import jax
import jax.numpy as jnp
from jax import lax
from jax.experimental import pallas as pl
from jax.experimental.pallas import tpu as pltpu
from jax.experimental.pallas import tpu_sc as plsc

N = 10000
E = 320000
D = 128
DE = 16

BLK = 128
NBLK = E // BLK
BPW = NBLK // 32
CHUNK = 13
NCH = BPW // CHUNK
ECH = CHUNK * BLK
NPAD = 10240
RPT = NPAD // 16


def _sc_scatter_body(idx_hbm, attr_hbm, out_hbm,
                     idx_v, av0, av1, tr0, tr1, shared, lsem, ssem):
    cid = lax.axis_index("c")
    sid = lax.axis_index("s")
    w = sid * 2 + cid
    row0 = sid * RPT

    @plsc.parallel_loop(0, RPT, step=1, unroll=8)
    def _z(i):
        tr0[i] = jnp.zeros((16,), jnp.float32)

    pltpu.sync_copy(tr0.at[pl.ds(0, RPT)], shared.at[pl.ds(row0, RPT)])
    plsc.subcore_barrier()

    base_b = w * BPW + jnp.minimum(w, 4)
    pltpu.sync_copy(idx_hbm.at[pl.ds(base_b, BPW), 0],
                    idx_v.at[pl.ds(0, BPW)])

    avs = [av0, av1]
    trs = [tr0, tr1]
    iota = lax.iota(jnp.int32, 16)
    fhi_vec = iota // 8
    flo_vec = iota % 8

    def transpose_chunk(av, tr, nblk):
        def blk_body(eblk, _):
            blk_vec = jnp.full((16,), eblk, jnp.int32)
            e0 = eblk * BLK

            @plsc.parallel_loop(0, BLK, step=1, unroll=16)
            def _t(elo):
                vals = plsc.load_gather(
                    av, [fhi_vec, blk_vec, flo_vec,
                         jnp.full((16,), elo, jnp.int32)])
                tr[e0 + elo] = vals

            return _

        lax.fori_loop(0, nblk, blk_body, 0)

    loads = [None] * NCH
    scatters = [None] * NCH
    loads[0] = pltpu.async_copy(
        attr_hbm.at[:, pl.ds(base_b, CHUNK)],
        avs[0].at[:, :, :, pl.ds(0, BLK)], lsem)
    for ch in range(NCH):
        buf = ch & 1
        if ch + 1 < NCH:
            loads[ch + 1] = pltpu.async_copy(
                attr_hbm.at[:, pl.ds(base_b + (ch + 1) * CHUNK, CHUNK)],
                avs[(ch + 1) & 1].at[:, :, :, pl.ds(0, BLK)], lsem)
        if ch >= 2:
            for dsc in scatters[ch - 2]:
                dsc.wait()
        loads[ch].wait()
        transpose_chunk(avs[buf], trs[buf], CHUNK)
        scatters[ch] = [
            pltpu.async_copy(trs[buf].at[pl.ds(b * BLK, BLK)],
                             shared.at[idx_v.at[ch * CHUNK + b]],
                             ssem, add=True)
            for b in range(CHUNK)
        ]
    for ch in (NCH - 2, NCH - 1):
        for dsc in scatters[ch]:
            dsc.wait()

    @pl.when(w < 4)
    def _tail():
        pltpu.sync_copy(idx_hbm.at[base_b + BPW, 0], idx_v.at[BPW])
        pltpu.sync_copy(attr_hbm.at[:, pl.ds(base_b + BPW, 1)],
                        avs[0].at[:, pl.ds(0, 1), :, pl.ds(0, BLK)])
        transpose_chunk(avs[0], trs[0], 1)
        pltpu.sync_copy(trs[0].at[pl.ds(0, BLK)],
                        shared.at[idx_v.at[BPW]], add=True)

    plsc.subcore_barrier()

    pltpu.sync_copy(shared.at[pl.ds(row0, RPT)],
                    out_hbm.at[cid, pl.ds(row0, RPT)])


_sc_scatter = functools.partial(
    pl.kernel,
    out_type=jax.ShapeDtypeStruct((2, NPAD, DE), jnp.float32),
    mesh=plsc.VectorSubcoreMesh(core_axis_name="c", subcore_axis_name="s"),
    scratch_types=[
        pltpu.VMEM((BPW + 1, BLK), jnp.int32),
        pltpu.VMEM((2, CHUNK, 8, BLK + 1), jnp.float32),
        pltpu.VMEM((2, CHUNK, 8, BLK + 1), jnp.float32),
        pltpu.VMEM((ECH, DE), jnp.float32),
        pltpu.VMEM((ECH, DE), jnp.float32),
        pltpu.VMEM_SHARED((NPAD, DE), jnp.float32),
        pltpu.SemaphoreType.DMA,
        pltpu.SemaphoreType.DMA,
    ],
    compiler_params=pltpu.CompilerParams(
        use_tc_tiling_on_sc=False, needs_layout_passes=False),
)(_sc_scatter_body)


def _tc_mlp_body(x_ref, p0_ref, p1_ref, w1x_ref, w1a_ref, b1_ref, w2_ref,
                 b2_ref, g_ref, bt_ref, o_ref):
    x = x_ref[...]
    pp = p0_ref[...] + p1_ref[...]
    ha = jnp.dot(pp, w1a_ref[...], preferred_element_type=jnp.float32)
    ha = ha.reshape(pp.shape[0], 8, D).reshape(x.shape[0], D)
    h = (jnp.dot(x, w1x_ref[...], preferred_element_type=jnp.float32)
         + ha + b1_ref[...])
    h = h * jax.nn.sigmoid(h)
    h = jnp.dot(h, w2_ref[...], preferred_element_type=jnp.float32) + b2_ref[...]
    mu = jnp.mean(h, axis=-1, keepdims=True)
    var = jnp.mean((h - mu) ** 2, axis=-1, keepdims=True)
    h = (h - mu) * lax.rsqrt(var + 1e-5) * g_ref[...] + bt_ref[...]
    o_ref[...] = h + x


def _tc_mlp(x, p0, p1, w1x, w1a, b1, w2, b2, gamma, beta):
    rows = 5120
    grid = ((N + rows - 1) // rows,)
    prows = rows // 8
    full = lambda shape: pl.BlockSpec(shape, lambda i: (0, 0))
    return pl.pallas_call(
        _tc_mlp_body,
        grid=grid,
        in_specs=[
            pl.BlockSpec((rows, D), lambda i: (i, 0)),
            pl.BlockSpec((prows, D), lambda i: (i, 0)),
            pl.BlockSpec((prows, D), lambda i: (i + NPAD * DE // D // prows, 0)),
            full((D, D)),
            full((D, 8 * D)),
            full((1, D)),
            full((D, D)),
            full((1, D)),
            full((1, D)),
            full((1, D)),
        ],
        out_specs=pl.BlockSpec((rows, D), lambda i: (i, 0)),
        out_shape=jax.ShapeDtypeStruct((N, D), jnp.float32),
    )(x, p0, p1, w1x, w1a, b1, w2, b2, gamma, beta)


def kernel(x, edge_index, edge_attr, W1, b1, W2, b2, gamma, beta):
    idx_r = edge_index.reshape(2, NBLK, BLK).transpose(1, 0, 2)
    attr4 = edge_attr.T.reshape(2, 8, NBLK, BLK).transpose(0, 2, 1, 3)

    partials = _sc_scatter(idx_r, attr4)
    p_packed = partials.reshape(2 * NPAD * DE // D, D)

    w1x = W1[:D]
    w1a = W1[D:]
    w1ap = jnp.kron(jnp.eye(8, dtype=jnp.float32), w1a)
    return _tc_mlp(x, p_packed, p_packed, w1x, w1ap,
                   b1.reshape(1, D), W2, b2.reshape(1, D),
                   gamma.reshape(1, D), beta.reshape(1, D))
